# baseline (speedup 1.0000x reference)
import functools

import jax
import jax.numpy as jnp
from jax import lax
from jax.experimental import pallas as pl
from jax.experimental.pallas import tpu as pltpu
from jax.experimental.pallas import tpu_sc as plsc

VOCAB = 1000000
D = 32
B = 4096
H = 200

NC = 2
NS = 16
NW = NC * NS

NCOL_FULL = VOCAB // 128
TAIL = VOCAB - NCOL_FULL * 128
NLINE = VOCAB // 4

NGRP = H * (B // 128)
GRP_PER_W = NGRP // NW

_PARAMS = pltpu.CompilerParams(
    use_tc_tiling_on_sc=True, needs_layout_passes=False
)
_MESH = plsc.VectorSubcoreMesh(core_axis_name="c", subcore_axis_name="s")


def _transpose_block(src_v, dst_v, iota, nrow):
    for r in range(nrow):
        for q in range(4):
            col = jnp.full((16,), 4 * r + q, jnp.int32)
            for half in range(2):
                rows = iota + (16 * half)
                v = plsc.load_gather(src_v, [rows, col])
                dst_v[r, pl.ds(q * 32 + 16 * half, 16)] = v


def _tab_body(tab_hbm, scr_hbm, in_v, tail_v, out_v, sem):
    wid = lax.axis_index("s") * NC + lax.axis_index("c")
    base, rem = NCOL_FULL // NW, NCOL_FULL % NW
    start = wid * base + jnp.minimum(wid, rem)
    count = base + jnp.where(wid < rem, 1, 0)
    iota = lax.broadcasted_iota(jnp.int32, (16,), 0)

    def col(i, carry):
        c = start + i
        cps = [
            pltpu.async_copy(
                tab_hbm.at[pl.ds(jb * 8, 8), pl.ds(c * 128, 128)],
                in_v.at[pl.ds(jb * 8, 8), :],
                sem,
            )
            for jb in range(4)
        ]
        for cp in cps:
            cp.wait()
        _transpose_block(in_v, out_v, iota, 32)
        pltpu.sync_copy(out_v, scr_hbm.at[pl.ds(c * 32, 32), :])
        return carry

    lax.fori_loop(0, count, col, 0)

    @pl.when(wid == NW - 1)
    def _tail():
        cps = [
            pltpu.async_copy(
                tab_hbm.at[pl.ds(jb * 8, 8), pl.ds(NCOL_FULL * 128, TAIL)],
                tail_v.at[pl.ds(jb * 8, 8), :],
                sem,
            )
            for jb in range(4)
        ]
        for cp in cps:
            cp.wait()
        _transpose_block(tail_v, out_v, iota, TAIL // 4)
        pltpu.sync_copy(
            out_v.at[pl.ds(0, TAIL // 4), :],
            scr_hbm.at[pl.ds(NCOL_FULL * 32, TAIL // 4), :],
        )


@functools.partial(
    pl.kernel,
    mesh=_MESH,
    out_type=jax.ShapeDtypeStruct((NLINE, 128), jnp.float32),
    scratch_types=[
        pltpu.VMEM((32, 128), jnp.float32),
        pltpu.VMEM((32, TAIL), jnp.float32),
        pltpu.VMEM((32, 128), jnp.float32),
        pltpu.SemaphoreType.DMA,
    ],
    compiler_params=_PARAMS,
)
def _tab_relayout(tab_hbm, scr_hbm, in_v, tail_v, out_v, sem):
    _tab_body(tab_hbm, scr_hbm, in_v, tail_v, out_v, sem)


def _gather_body(xt_hbm, scr_hbm, out_hbm, idx_v, lines_v, lrows_v, tile_v, sem):
    wid = lax.axis_index("s") * NC + lax.axis_index("c")
    iota = lax.broadcasted_iota(jnp.int32, (16,), 0)

    def group(t, carry):
        g = wid * GRP_PER_W + t
        h = g // (B // 128)
        bb = g % (B // 128)
        pltpu.sync_copy(xt_hbm.at[h, pl.ds(bb * 128, 128)], idx_v)
        for k in range(8):
            iv = idx_v[pl.ds(16 * k, 16)]
            lines_v[pl.ds(16 * k, 16)] = lax.shift_right_logical(iv, 2)
        pltpu.async_copy(scr_hbm.at[lines_v], lrows_v, sem).wait()
        lane_col = []
        for k in range(8):
            iv = idx_v[pl.ds(16 * k, 16)]
            colb = lax.shift_left(jnp.bitwise_and(iv, 3), 5)
            lane_col.append((iota + 16 * k, colb))
        for j in range(D):
            for k in range(8):
                rows, colb = lane_col[k]
                v = plsc.load_gather(lrows_v, [rows, colb + j])
                tile_v[j, pl.ds(16 * k, 16)] = v
        for jb in range(4):
            pltpu.sync_copy(
                tile_v.at[pl.ds(jb * 8, 8), :],
                out_hbm.at[h, pl.ds(jb * 8, 8), pl.ds(bb * 128, 128)],
            )
        return carry

    lax.fori_loop(0, GRP_PER_W, group, 0)


@functools.partial(
    pl.kernel,
    mesh=_MESH,
    out_type=jax.ShapeDtypeStruct((H, D, B), jnp.float32),
    scratch_types=[
        pltpu.VMEM((128,), jnp.int32),
        pltpu.VMEM((128,), jnp.int32),
        pltpu.VMEM((128, 128), jnp.float32),
        pltpu.VMEM((D, 128), jnp.float32),
        pltpu.SemaphoreType.DMA,
    ],
    compiler_params=_PARAMS,
)
def _emb_gather(xt_hbm, scr_hbm, out_hbm, idx_v, lines_v, lrows_v, tile_v, sem):
    _gather_body(xt_hbm, scr_hbm, out_hbm, idx_v, lines_v, lrows_v, tile_v, sem)


def kernel(X, table):
    tab_t = table.T
    x_t = X.astype(jnp.int32).T
    scratch = _tab_relayout(tab_t)
    out3 = _emb_gather(x_t, scratch)
    return jnp.transpose(out3, (2, 0, 1))


# speedup vs baseline: 1.7332x; 1.7332x over previous
import functools

import jax
import jax.numpy as jnp
from jax import lax
from jax.experimental import pallas as pl
from jax.experimental.pallas import tpu as pltpu
from jax.experimental.pallas import tpu_sc as plsc

VOCAB = 1000000
D = 32
B = 4096
H = 200

NC = 2
NS = 16
NW = NC * NS

NCOL = VOCAB // 128
COLS_PER_W = NCOL // NW
COL_EXTRA = NCOL - COLS_PER_W * NW
TAIL = VOCAB - NCOL * 128
NLINE = VOCAB // 4

NBLK = B // 128
NGRP = H * NBLK
GRP_PER_W = NGRP // NW

_PARAMS = pltpu.CompilerParams(
    use_tc_tiling_on_sc=True, needs_layout_passes=False
)
_MESH = plsc.VectorSubcoreMesh(core_axis_name="c", subcore_axis_name="s")


def _transpose_block(src_v, dst_v, iota, nrow):
    for r0 in range(0, nrow, 2):
        vals = []
        for r in (r0, r0 + 1):
            if r >= nrow:
                continue
            for q in range(4):
                col = jnp.full((16,), 4 * r + q, jnp.int32)
                for half in range(2):
                    v = plsc.load_gather(src_v, [iota + 16 * half, col])
                    vals.append((r, q, half, v))
        for r, q, half, v in vals:
            dst_v[r, pl.ds(q * 32 + 16 * half, 16)] = v


def _tab_body(tab_hbm, scr_hbm, in_v, out_v, tail_v, sem_ld, sem_st):
    wid = lax.axis_index("s") * NC + lax.axis_index("c")
    iota = lax.broadcasted_iota(jnp.int32, (16,), 0)

    def col_of(k):
        return wid + k * NW

    def issue_loads(c, par):
        for jb in range(4):
            pltpu.async_copy(
                tab_hbm.at[pl.ds(jb * 8, 8), pl.ds(c * 128, 128)],
                in_v.at[par, pl.ds(jb * 8, 8), :],
                sem_ld,
            )

    def wait_loads(c, par):
        for jb in range(4):
            pltpu.make_async_copy(
                tab_hbm.at[pl.ds(jb * 8, 8), pl.ds(c * 128, 128)],
                in_v.at[par, pl.ds(jb * 8, 8), :],
                sem_ld,
            ).wait()

    def store(c, par):
        pltpu.async_copy(
            out_v.at[par], scr_hbm.at[pl.ds(c * 32, 32), :], sem_st
        )

    def wait_store(c, par):
        pltpu.make_async_copy(
            out_v.at[par], scr_hbm.at[pl.ds(c * 32, 32), :], sem_st
        ).wait()

    issue_loads(col_of(0), 0)
    issue_loads(col_of(1), 1)

    def body(k2, carry):
        for par in range(2):
            k = 2 * k2 + par
            c = col_of(k)
            wait_loads(c, par)

            @pl.when(k2 > 0)
            def _():
                wait_store(c, par)

            _transpose_block(in_v.at[par], out_v.at[par], iota, 32)
            store(c, par)

            @pl.when(k2 < (COLS_PER_W // 2 - 1))
            def _():
                issue_loads(col_of(k + 2), par)

        return carry

    lax.fori_loop(0, COLS_PER_W // 2, body, 0)
    wait_store(col_of(COLS_PER_W - 2), 0)
    wait_store(col_of(COLS_PER_W - 1), 1)

    @pl.when(wid < COL_EXTRA)
    def _extra():
        c = COLS_PER_W * NW + wid
        issue_loads(c, 0)
        wait_loads(c, 0)
        _transpose_block(in_v.at[0], out_v.at[0], iota, 32)
        pltpu.sync_copy(out_v.at[0], scr_hbm.at[pl.ds(c * 32, 32), :])

    @pl.when(wid == NW - 1)
    def _tail():
        for jb in range(4):
            pltpu.async_copy(
                tab_hbm.at[pl.ds(jb * 8, 8), pl.ds(NCOL * 128, TAIL)],
                tail_v.at[pl.ds(jb * 8, 8), :],
                sem_ld,
            )
        for jb in range(4):
            pltpu.make_async_copy(
                tab_hbm.at[pl.ds(jb * 8, 8), pl.ds(NCOL * 128, TAIL)],
                tail_v.at[pl.ds(jb * 8, 8), :],
                sem_ld,
            ).wait()
        _transpose_block(tail_v, out_v.at[0], iota, TAIL // 4)
        pltpu.sync_copy(
            out_v.at[0, pl.ds(0, TAIL // 4), :],
            scr_hbm.at[pl.ds(NCOL * 32, TAIL // 4), :],
        )


@functools.partial(
    pl.kernel,
    mesh=_MESH,
    out_type=jax.ShapeDtypeStruct((NLINE, 128), jnp.float32),
    scratch_types=[
        pltpu.VMEM((2, 32, 128), jnp.float32),
        pltpu.VMEM((2, 32, 128), jnp.float32),
        pltpu.VMEM((32, TAIL), jnp.float32),
        pltpu.SemaphoreType.DMA,
        pltpu.SemaphoreType.DMA,
    ],
    compiler_params=_PARAMS,
)
def _tab_relayout(tab_hbm, scr_hbm, in_v, out_v, tail_v, sem_ld, sem_st):
    _tab_body(tab_hbm, scr_hbm, in_v, out_v, tail_v, sem_ld, sem_st)


def _gather_body(
    xt_hbm, scr_hbm, out_hbm, idx_v, lines_v, lrows_v, tile_v, sem_g, sem_st
):
    wid = lax.axis_index("s") * NC + lax.axis_index("c")
    iota = lax.broadcasted_iota(jnp.int32, (16,), 0)

    def hb_of(t):
        g = wid * GRP_PER_W + t
        return g // NBLK, g % NBLK

    def prefetch(t, par):
        h, bb = hb_of(t)
        pltpu.sync_copy(xt_hbm.at[h, pl.ds(bb * 128, 128)], idx_v.at[par])
        for k in range(8):
            iv = idx_v[par, pl.ds(16 * k, 16)]
            lines_v[par, pl.ds(16 * k, 16)] = lax.shift_right_logical(iv, 2)
        pltpu.async_copy(
            scr_hbm.at[lines_v.at[par]], lrows_v.at[par], sem_g
        )

    def wait_gather(par):
        pltpu.make_async_copy(
            scr_hbm.at[pl.ds(0, 128), :], lrows_v.at[par], sem_g
        ).wait()

    def store_tile(t, par):
        h, bb = hb_of(t)
        pltpu.async_copy(
            tile_v.at[par],
            out_hbm.at[h, pl.ds(0, D), pl.ds(bb * 128, 128)],
            sem_st,
        )

    def wait_store(t, par):
        h, bb = hb_of(t)
        pltpu.make_async_copy(
            tile_v.at[par],
            out_hbm.at[h, pl.ds(0, D), pl.ds(bb * 128, 128)],
            sem_st,
        ).wait()

    def extract(par):
        lane_col = []
        for k in range(8):
            iv = idx_v[par, pl.ds(16 * k, 16)]
            colb = lax.shift_left(jnp.bitwise_and(iv, 3), 5)
            lane_col.append((iota + 16 * k, colb))
        for j0 in range(0, D, 2):
            vals = []
            for j in (j0, j0 + 1):
                for k in range(8):
                    rows, colb = lane_col[k]
                    v = plsc.load_gather(lrows_v.at[par], [rows, colb + j])
                    vals.append((j, k, v))
            for j, k, v in vals:
                tile_v[par, j, pl.ds(16 * k, 16)] = v

    prefetch(0, 0)
    prefetch(1, 1)

    def body(t2, carry):
        for par in range(2):
            t = 2 * t2 + par
            wait_gather(par)

            @pl.when(t2 > 0)
            def _():
                wait_store(t, par)

            extract(par)
            store_tile(t, par)

            @pl.when(t2 < (GRP_PER_W // 2 - 1))
            def _():
                prefetch(t + 2, par)

        return carry

    lax.fori_loop(0, GRP_PER_W // 2, body, 0)
    wait_store(GRP_PER_W - 2, 0)
    wait_store(GRP_PER_W - 1, 1)


@functools.partial(
    pl.kernel,
    mesh=_MESH,
    out_type=jax.ShapeDtypeStruct((H, D, B), jnp.float32),
    scratch_types=[
        pltpu.VMEM((2, 128), jnp.int32),
        pltpu.VMEM((2, 128), jnp.int32),
        pltpu.VMEM((2, 128, 128), jnp.float32),
        pltpu.VMEM((2, D, 128), jnp.float32),
        pltpu.SemaphoreType.DMA,
        pltpu.SemaphoreType.DMA,
    ],
    compiler_params=_PARAMS,
)
def _emb_gather(
    xt_hbm, scr_hbm, out_hbm, idx_v, lines_v, lrows_v, tile_v, sem_g, sem_st
):
    _gather_body(
        xt_hbm, scr_hbm, out_hbm, idx_v, lines_v, lrows_v, tile_v, sem_g, sem_st
    )


def kernel(X, table):
    tab_t = table.T
    x_t = X.astype(jnp.int32).T
    scratch = _tab_relayout(tab_t)
    out3 = _emb_gather(x_t, scratch)
    return jnp.transpose(out3, (2, 0, 1))


# speedup vs baseline: 1.8734x; 1.0809x over previous
import functools

import jax
import jax.numpy as jnp
from jax import lax
from jax.experimental import pallas as pl
from jax.experimental.pallas import tpu as pltpu
from jax.experimental.pallas import tpu_sc as plsc

VOCAB = 1000000
D = 32
B = 4096
H = 200

NC = 2
NS = 16
NW = NC * NS

NCOL = VOCAB // 128
COLS_PER_W = NCOL // NW
COL_EXTRA = NCOL - COLS_PER_W * NW
TAIL = VOCAB - NCOL * 128
RPL = 2
NLINE = VOCAB // RPL
LPC = 128 // RPL

NBLK = B // 128
GRP_PER_W = H
NBUF = 4

WIN = 8

_PARAMS = pltpu.CompilerParams(
    use_tc_tiling_on_sc=True, needs_layout_passes=False
)
_MESH = plsc.VectorSubcoreMesh(core_axis_name="c", subcore_axis_name="s")


def _windowed(ops):
    pend = []
    for emit, commit in ops:
        pend.append((commit, emit()))
        if len(pend) > WIN:
            c, v = pend.pop(0)
            c(v)
    for c, v in pend:
        c(v)


def _transpose_block(src_v, dst_v, iota, nlane):
    def ops():
        for r in range(nlane // RPL):
            for q in range(RPL):
                col = jnp.full((16,), RPL * r + q, jnp.int32)
                for half in range(2):
                    def emit(col=col, half=half):
                        return plsc.load_gather(src_v, [iota + 16 * half, col])

                    def commit(v, r=r, q=q, half=half):
                        dst_v[r, pl.ds(q * 32 + 16 * half, 16)] = v

                    yield emit, commit

    _windowed(ops())


def _tab_body(tab_hbm, scr_hbm, in_v, out_v, tail_v, sem_ld, sem_st):
    wid = lax.axis_index("s") * NC + lax.axis_index("c")
    iota = lax.broadcasted_iota(jnp.int32, (16,), 0)

    def col_of(k):
        return wid + k * NW

    def issue_loads(c, par):
        for jb in range(4):
            pltpu.async_copy(
                tab_hbm.at[pl.ds(jb * 8, 8), pl.ds(c * 128, 128)],
                in_v.at[par, pl.ds(jb * 8, 8), :],
                sem_ld,
            )

    def wait_loads(c, par):
        for jb in range(4):
            pltpu.make_async_copy(
                tab_hbm.at[pl.ds(jb * 8, 8), pl.ds(c * 128, 128)],
                in_v.at[par, pl.ds(jb * 8, 8), :],
                sem_ld,
            ).wait()

    def store(c, par):
        pltpu.async_copy(
            out_v.at[par], scr_hbm.at[pl.ds(c * LPC, LPC), :], sem_st
        )

    def wait_store(c, par):
        pltpu.make_async_copy(
            out_v.at[par], scr_hbm.at[pl.ds(c * LPC, LPC), :], sem_st
        ).wait()

    issue_loads(col_of(0), 0)
    issue_loads(col_of(1), 1)

    def body(k2, carry):
        for par in range(2):
            k = 2 * k2 + par
            c = col_of(k)
            wait_loads(c, par)

            @pl.when(k2 > 0)
            def _():
                wait_store(c, par)

            _transpose_block(in_v.at[par], out_v.at[par], iota, 128)
            store(c, par)

            @pl.when(k2 < (COLS_PER_W // 2 - 1))
            def _():
                issue_loads(col_of(k + 2), par)

        return carry

    lax.fori_loop(0, COLS_PER_W // 2, body, 0)
    wait_store(col_of(COLS_PER_W - 2), 0)
    wait_store(col_of(COLS_PER_W - 1), 1)

    @pl.when(wid < COL_EXTRA)
    def _extra():
        c = COLS_PER_W * NW + wid
        issue_loads(c, 0)
        wait_loads(c, 0)
        _transpose_block(in_v.at[0], out_v.at[0], iota, 128)
        pltpu.sync_copy(out_v.at[0], scr_hbm.at[pl.ds(c * LPC, LPC), :])

    @pl.when(wid == NW - 1)
    def _tail():
        for jb in range(4):
            pltpu.async_copy(
                tab_hbm.at[pl.ds(jb * 8, 8), pl.ds(NCOL * 128, TAIL)],
                tail_v.at[pl.ds(jb * 8, 8), :],
                sem_ld,
            )
        for jb in range(4):
            pltpu.make_async_copy(
                tab_hbm.at[pl.ds(jb * 8, 8), pl.ds(NCOL * 128, TAIL)],
                tail_v.at[pl.ds(jb * 8, 8), :],
                sem_ld,
            ).wait()
        _transpose_block(tail_v, out_v.at[0], iota, TAIL)
        pltpu.sync_copy(
            out_v.at[0, pl.ds(0, TAIL // RPL), :],
            scr_hbm.at[pl.ds(NCOL * LPC, TAIL // RPL), :],
        )


@functools.partial(
    pl.kernel,
    mesh=_MESH,
    out_type=jax.ShapeDtypeStruct((NLINE, 128), jnp.float32),
    scratch_types=[
        pltpu.VMEM((2, 32, 128), jnp.float32),
        pltpu.VMEM((2, LPC, 128), jnp.float32),
        pltpu.VMEM((32, TAIL), jnp.float32),
        pltpu.SemaphoreType.DMA,
        pltpu.SemaphoreType.DMA,
    ],
    compiler_params=_PARAMS,
)
def _tab_relayout(tab_hbm, scr_hbm, in_v, out_v, tail_v, sem_ld, sem_st):
    _tab_body(tab_hbm, scr_hbm, in_v, out_v, tail_v, sem_ld, sem_st)


def _gather_body(
    xt_hbm, scr_hbm, out_hbm, idx_v, lines_v, lrows_v, tile_v, sem_g, sem_st
):
    wid = lax.axis_index("s") * NC + lax.axis_index("c")
    iota = lax.broadcasted_iota(jnp.int32, (16,), 0)

    pltpu.sync_copy(xt_hbm.at[:, pl.ds(wid * 128, 128)], idx_v)

    def prefetch(t, buf):
        for k in range(8):
            iv = idx_v[t, pl.ds(16 * k, 16)]
            lines_v[buf, pl.ds(16 * k, 16)] = lax.shift_right_logical(iv, 1)
        pltpu.async_copy(scr_hbm.at[lines_v.at[buf]], lrows_v.at[buf], sem_g)

    def wait_gather(buf):
        pltpu.make_async_copy(
            scr_hbm.at[pl.ds(0, 128), :], lrows_v.at[buf], sem_g
        ).wait()

    def store_tile(t, par):
        pltpu.async_copy(
            tile_v.at[par],
            out_hbm.at[t, pl.ds(0, D), pl.ds(wid * 128, 128)],
            sem_st,
        )

    def wait_store(t, par):
        pltpu.make_async_copy(
            tile_v.at[par],
            out_hbm.at[t, pl.ds(0, D), pl.ds(wid * 128, 128)],
            sem_st,
        ).wait()

    def extract(t, buf, par):
        lane_col = []
        for k in range(8):
            iv = idx_v[t, pl.ds(16 * k, 16)]
            colb = lax.shift_left(jnp.bitwise_and(iv, RPL - 1), 5)
            lane_col.append((iota + 16 * k, colb))

        def ops():
            for j in range(D):
                for k in range(8):
                    rows, colb = lane_col[k]

                    def emit(rows=rows, colb=colb, j=j):
                        return plsc.load_gather(
                            lrows_v.at[buf], [rows, colb + j]
                        )

                    def commit(v, j=j, k=k):
                        tile_v[par, j, pl.ds(16 * k, 16)] = v

                    yield emit, commit

        _windowed(ops())

    for b in range(NBUF):
        prefetch(b, b)

    def body(t2, carry):
        for par in range(NBUF):
            t = NBUF * t2 + par
            wait_gather(par)

            @pl.when(t2 > 0)
            def _():
                wait_store(t, par)

            extract(t, par, par)
            store_tile(t, par)

            @pl.when(t2 < (GRP_PER_W // NBUF - 1))
            def _():
                prefetch(t + NBUF, par)

        return carry

    lax.fori_loop(0, GRP_PER_W // NBUF, body, 0)
    for par in range(NBUF):
        wait_store(GRP_PER_W - NBUF + par, par)


@functools.partial(
    pl.kernel,
    mesh=_MESH,
    out_type=jax.ShapeDtypeStruct((H, D, B), jnp.float32),
    scratch_types=[
        pltpu.VMEM((H, 128), jnp.int32),
        pltpu.VMEM((NBUF, 128), jnp.int32),
        pltpu.VMEM((NBUF, 128, 128), jnp.float32),
        pltpu.VMEM((NBUF, D, 128), jnp.float32),
        pltpu.SemaphoreType.DMA,
        pltpu.SemaphoreType.DMA,
    ],
    compiler_params=_PARAMS,
)
def _emb_gather(
    xt_hbm, scr_hbm, out_hbm, idx_v, lines_v, lrows_v, tile_v, sem_g, sem_st
):
    _gather_body(
        xt_hbm, scr_hbm, out_hbm, idx_v, lines_v, lrows_v, tile_v, sem_g, sem_st
    )


def kernel(X, table):
    tab_t = table.T
    x_t = X.astype(jnp.int32).T
    scratch = _tab_relayout(tab_t)
    out3 = _emb_gather(x_t, scratch)
    return jnp.transpose(out3, (2, 0, 1))
